# Initial kernel scaffold; baseline (speedup 1.0000x reference)
import jax
import jax.numpy as jnp
from jax.experimental import pallas as pl


def kernel(x, s, affine_w, affine_b, weight, bias):
    raise NotImplementedError("write your pallas kernel here")


# speedup vs baseline: 18.5729x; 18.5729x over previous
import functools

import numpy as np
import jax
import jax.numpy as jnp
from jax import lax
from jax.experimental import pallas as pl
from jax.experimental.pallas import tpu as pltpu
from jax.experimental.pallas import tpu_sc as plsc

_RES_MIN, _RES_MAX = 16, 256
_L, _T = 16, 65536
_STYLE_DIM = 512
_PRIME = np.uint32(2654435761)

_NC, _NS = 2, 16
_NW = _NC * _NS
_CHUNK = 128

_R1MAX = 257
_W_PAD = 258
_M = 256 * _W_PAD
_ROWS = 66568

_HIGH = jax.lax.Precision.HIGHEST


def _build_static_plan():
    growth = float(np.exp((np.log(_RES_MAX) - np.log(_RES_MIN)) / (_L - 1)))
    res = [int(np.floor(_RES_MIN * growth ** l)) for l in range(_L)]
    idx_parts, offs = [], []
    a_all = np.zeros((_L, 256, _R1MAX), np.float32)
    off = 0
    for l, r in enumerate(res):
        r1 = r + 1
        ax = np.arange(r1, dtype=np.uint32)
        gi = (ax[None, :] ^ (ax[:, None] * _PRIME)) % np.uint32(_T)
        idx_parts.append((np.int32(l * _T) + gi.astype(np.int32)).reshape(-1))
        offs.append(off)
        off += r1 * r1
        x = (np.arange(256, dtype=np.float32) + np.float32(0.5)) / np.float32(256)
        pos = x * np.float32(r)
        p0 = np.floor(pos)
        w = pos - p0
        p0 = p0.astype(np.int64)
        a = np.zeros((256, _R1MAX), np.float32)
        a[np.arange(256), p0] = 1.0 - w
        a[np.arange(256), p0 + 1] = w
        a_all[l] = a
    k_total = off
    k_pad = ((k_total + _NW * _CHUNK - 1) // (_NW * _CHUNK)) * (_NW * _CHUNK)
    idx = np.zeros((k_pad,), np.int32)
    idx[:k_total] = np.concatenate(idx_parts)
    return (res, offs, idx.reshape(_NW, k_pad // (_NW * _CHUNK), _CHUNK),
            a_all, k_pad)


_RES, _OFFS, _IDX2D, _A_ALL, _K_PAD = _build_static_plan()
_B_PER_W = _K_PAD // _NW
_NCHUNKS_W = _B_PER_W // _CHUNK


def _sc_gather(tbl, idx2d):
    mesh = plsc.VectorSubcoreMesh(core_axis_name="c", subcore_axis_name="s")

    @functools.partial(
        pl.kernel, mesh=mesh,
        compiler_params=pltpu.CompilerParams(use_tc_tiling_on_sc=False),
        out_type=jax.ShapeDtypeStruct((_K_PAD, 8), jnp.float32),
        scratch_types=[
            pltpu.VMEM((_NCHUNKS_W, _CHUNK), jnp.int32),
            pltpu.VMEM((_B_PER_W, 8), jnp.float32),
            pltpu.SemaphoreType.DMA,
        ],
    )
    def k(tbl_hbm, idx_hbm, out_hbm, idx_v, rows_v, sem):
        wid = lax.axis_index("s") * _NC + lax.axis_index("c")
        pltpu.sync_copy(idx_hbm.at[wid], idx_v)

        def body(j, carry):
            pltpu.async_copy(
                tbl_hbm.at[idx_v.at[j]],
                rows_v.at[pl.ds(j * _CHUNK, _CHUNK)],
                sem,
            ).wait()
            return carry

        lax.fori_loop(0, _NCHUNKS_W, body, 0)
        pltpu.sync_copy(rows_v, out_hbm.at[pl.ds(wid * _B_PER_W, _B_PER_W)])

    return k(tbl, idx2d)


def _interp_kernel(ay_ref, gt_ref, axt_ref, out_ref):
    ay = ay_ref[0]
    axt = axt_ref[0]
    for f in range(8):
        tmp = lax.dot(ay, gt_ref[0, f], precision=_HIGH)
        out_ref[0, f] = lax.dot(tmp, axt, precision=_HIGH)


def _interp(gt_all, ay_all, axt_all):
    return pl.pallas_call(
        _interp_kernel,
        grid=(_L,),
        in_specs=[
            pl.BlockSpec((1, 256, _R1MAX), lambda l: (l, 0, 0)),
            pl.BlockSpec((1, 8, _R1MAX, _R1MAX), lambda l: (l, 0, 0, 0)),
            pl.BlockSpec((1, _R1MAX, 256), lambda l: (l, 0, 0)),
        ],
        out_specs=pl.BlockSpec((1, 8, 256, 256), lambda l: (l, 0, 0, 0)),
        out_shape=jax.ShapeDtypeStruct((_L, 8, 256, 256), jnp.float32),
    )(ay_all, gt_all, axt_all)


def _conv_kernel(xf_ref, s_ref, awt_ref, ab_ref, wsq_ref, wgo_ref, bias_ref,
                 out_ref):
    sb = s_ref[pl.ds(pl.program_id(0), 1), :]
    sty = lax.dot(sb, awt_ref[...], precision=_HIGH) + ab_ref[...]
    dsum = lax.dot(sty * sty, wsq_ref[...], precision=_HIGH) + 1e-8
    dcol = jnp.transpose(lax.rsqrt(dsum))
    acc = jnp.zeros((3, _M), jnp.float32)
    for k in range(9):
        off = (k // 3) * _W_PAD + (k % 3)
        xs = xf_ref[0, :, pl.ds(off, _M)]
        wm = wgo_ref[k] * sty
        acc = acc + lax.dot_general(
            wm, xs, (((1,), (0,)), ((), ())), precision=_HIGH)
    out = acc * dcol + bias_ref[...]
    out_ref[0] = jnp.clip(out, -256.0, 256.0)


def _conv(xf, s, awt, ab2, wsq, wgo, bias_row):
    return pl.pallas_call(
        _conv_kernel,
        grid=(4,),
        in_specs=[
            pl.BlockSpec((1, 32, _ROWS), lambda b: (b, 0, 0)),
            pl.BlockSpec((4, _STYLE_DIM), lambda b: (0, 0)),
            pl.BlockSpec((_STYLE_DIM, 32), lambda b: (0, 0)),
            pl.BlockSpec((1, 32), lambda b: (0, 0)),
            pl.BlockSpec((32, 3), lambda b: (0, 0)),
            pl.BlockSpec((9, 3, 32), lambda b: (0, 0, 0)),
            pl.BlockSpec((3, 1), lambda b: (0, 0)),
        ],
        out_specs=pl.BlockSpec((1, 3, _M), lambda b: (b, 0, 0)),
        out_shape=jax.ShapeDtypeStruct((4, 3, _M), jnp.float32),
    )(xf, s, awt, ab2, wsq, wgo, bias_row)


def kernel(x, s, affine_w, affine_b, weight, bias):
    b = x.shape[0]
    tbl = (x.reshape(b, _L, _T, 2).transpose(1, 2, 0, 3)
           .reshape(_L * _T, b * 2))
    idx2d = jnp.asarray(_IDX2D)

    rows = _sc_gather(tbl, idx2d)

    gts = []
    for l, r in enumerate(_RES):
        r1 = r + 1
        g = rows[_OFFS[l]:_OFFS[l] + r1 * r1].reshape(r1, r1, 8)
        g = g.transpose(2, 0, 1)
        g = jnp.pad(g, ((0, 0), (0, _R1MAX - r1), (0, _R1MAX - r1)))
        gts.append(g)
    gt_all = jnp.stack(gts)
    ay_all = jnp.asarray(_A_ALL)
    axt_all = jnp.asarray(np.ascontiguousarray(_A_ALL.transpose(0, 2, 1)))

    feats = _interp(gt_all, ay_all, axt_all)

    ci = (feats.reshape(_L, b, 2, 256, 256)
          .transpose(1, 0, 2, 3, 4).reshape(b, _L * 2, 256, 256))
    xp = jnp.pad(ci, ((0, 0), (0, 0), (1, 1), (1, 1)))
    xf = xp.reshape(b, _L * 2, _W_PAD * _W_PAD)
    xf = jnp.pad(xf, ((0, 0), (0, 0), (0, _ROWS - _W_PAD * _W_PAD)))

    awt = (affine_w * np.float32(1.0 / np.sqrt(_STYLE_DIM))).T
    ab2 = affine_b.reshape(1, _L * 2)
    wscale = np.float32(1.0 / np.sqrt(weight.shape[1] * weight.shape[2]
                                      * weight.shape[3]))
    wg = weight * wscale
    wgo = wg.transpose(2, 3, 0, 1).reshape(9, 3, 32)
    wsq = jnp.sum(wg * wg, axis=(2, 3)).T
    bias_col = bias.reshape(3, 1)

    out = _conv(xf, s, awt, ab2, wsq, wgo, bias_col)
    y = out.reshape(b, 3, 256, _W_PAD)[:, :, :, :256]
    return y
